# Initial kernel scaffold; baseline (speedup 1.0000x reference)
import jax
import jax.numpy as jnp
from jax.experimental import pallas as pl


def kernel(query, memory_states, k):
    raise NotImplementedError("write your pallas kernel here")


# speedup vs baseline: 1.3804x; 1.3804x over previous
import functools

import jax
import jax.numpy as jnp
from jax import lax
from jax.experimental import pallas as pl
from jax.experimental.pallas import tpu as pltpu
from jax.experimental.pallas import tpu_sc as plsc

K = 8
QB = 256
NB = 2048
_NEG_INF = float("-inf")
_BIG_I32 = 2**30

_SC_CORES = 2
_SC_SUBCORES = 16
_SC_WORKERS = _SC_CORES * _SC_SUBCORES


def _extract_topk(s, col, base, kk):
    vals, idxs = [], []
    for _ in range(kk):
        mx = jnp.max(s, axis=1, keepdims=True)
        t = jnp.where(s == mx, col, _BIG_I32)
        pos = jnp.min(t, axis=1, keepdims=True)
        s = jnp.where(col == pos, _NEG_INF, s)
        vals.append(mx)
        idxs.append(pos + base)
    return jnp.concatenate(vals, axis=1), jnp.concatenate(idxs, axis=1)


def _topk_body(n_real, qb, q_ref, m_ref, vals_ref, idx_ref, runv_ref, runi_ref):
    ni = pl.program_id(1)
    nblocks = pl.num_programs(1)

    @pl.when(ni == 0)
    def _init():
        runv_ref[...] = jnp.full((qb, K), _NEG_INF, jnp.float32)
        runi_ref[...] = jnp.zeros((qb, K), jnp.int32)

    q = q_ref[...]
    qn = q / jnp.maximum(
        jnp.sqrt(jnp.sum(q * q, axis=1, keepdims=True)), 1e-12)
    m = m_ref[...]
    mn = m / jnp.maximum(
        jnp.sqrt(jnp.sum(m * m, axis=1, keepdims=True)), 1e-12)
    s = lax.dot_general(qn, mn, (((1,), (1,)), ((), ())))

    base = ni * NB
    col = lax.broadcasted_iota(jnp.int32, s.shape, 1)
    s = jnp.where(col + base < n_real, s, _NEG_INF)

    bv, bi = _extract_topk(s, col, base, K)

    cv = jnp.concatenate([runv_ref[...], bv], axis=1)
    ci = jnp.concatenate([runi_ref[...], bi], axis=1)
    col2 = lax.broadcasted_iota(jnp.int32, cv.shape, 1)
    nv, nidx = [], []
    for _ in range(K):
        mx = jnp.max(cv, axis=1, keepdims=True)
        t = jnp.where(cv == mx, col2, _BIG_I32)
        pos = jnp.min(t, axis=1, keepdims=True)
        sel = col2 == pos
        nidx.append(jnp.sum(jnp.where(sel, ci, 0), axis=1, keepdims=True))
        cv = jnp.where(sel, _NEG_INF, cv)
        nv.append(mx)
    runv_ref[...] = jnp.concatenate(nv, axis=1)
    runi_ref[...] = jnp.concatenate(nidx, axis=1)

    @pl.when(ni == nblocks - 1)
    def _emit():
        vals_ref[...] = runv_ref[...]
        idx_ref[...] = runi_ref[...]


def _topk_scores(query, memory_states):
    q, d = query.shape
    n = memory_states.shape[0]
    qb = QB if q % QB == 0 else (8 if q % 8 == 0 else 1)
    n_pad = ((n + NB - 1) // NB) * NB
    if n_pad != n:
        memory_states = jnp.pad(memory_states, ((0, n_pad - n), (0, 0)))
    grid = (q // qb, n_pad // NB)
    vals, idx = pl.pallas_call(
        functools.partial(_topk_body, n, qb),
        grid=grid,
        in_specs=[
            pl.BlockSpec((qb, d), lambda qi, ni: (qi, 0)),
            pl.BlockSpec((NB, d), lambda qi, ni: (ni, 0)),
        ],
        out_specs=[
            pl.BlockSpec((qb, K), lambda qi, ni: (qi, 0)),
            pl.BlockSpec((qb, K), lambda qi, ni: (qi, 0)),
        ],
        out_shape=[
            jax.ShapeDtypeStruct((q, K), jnp.float32),
            jax.ShapeDtypeStruct((q, K), jnp.int32),
        ],
        scratch_shapes=[
            pltpu.VMEM((qb, K), jnp.float32),
            pltpu.VMEM((qb, K), jnp.int32),
        ],
    )(query, memory_states)
    return vals, idx


def _sc_gather(table, flat_idx):
    b, d = flat_idx.shape[0], table.shape[1]
    b_per_w = b // _SC_WORKERS
    mesh = plsc.VectorSubcoreMesh(core_axis_name="c", subcore_axis_name="s")

    @functools.partial(
        pl.kernel,
        mesh=mesh,
        out_type=jax.ShapeDtypeStruct((b, d), jnp.float32),
        scratch_types=[
            pltpu.VMEM((b_per_w,), jnp.int32),
            pltpu.VMEM((b_per_w, d), jnp.float32),
            pltpu.SemaphoreType.DMA,
        ],
    )
    def gather(table_hbm, idx_hbm, out_hbm, idx_v, rows_v, sem):
        wid = lax.axis_index("s") * _SC_CORES + lax.axis_index("c")
        base = wid * b_per_w
        pltpu.sync_copy(idx_hbm.at[pl.ds(base, b_per_w)], idx_v)
        pltpu.async_copy(table_hbm.at[idx_v], rows_v, sem).wait()
        pltpu.sync_copy(rows_v, out_hbm.at[pl.ds(base, b_per_w)])

    return gather(table, flat_idx)


def kernel(query, memory_states, k):
    if query.ndim == 1:
        query = query[None, :]
    q, d = query.shape
    vals, idx = _topk_scores(query, memory_states)
    flat_idx = idx.reshape(q * K)
    algn = 8 * _SC_WORKERS
    b_pad = ((q * K + algn - 1) // algn) * algn
    if b_pad != q * K:
        flat_idx = jnp.pad(flat_idx, (0, b_pad - q * K))
    rows = _sc_gather(memory_states, flat_idx)
    retrieved = rows[: q * K].reshape(q, K, d)
    return (retrieved, vals)


# speedup vs baseline: 2.4206x; 1.7535x over previous
import functools

import jax
import jax.numpy as jnp
from jax import lax
from jax.experimental import pallas as pl
from jax.experimental.pallas import tpu as pltpu
from jax.experimental.pallas import tpu_sc as plsc

K = 8
QB = 256
NB = 4096
_NEG_INF = float("-inf")
_BIGF = 1e9

_SC_CORES = 2
_SC_SUBCORES = 16
_SC_WORKERS = _SC_CORES * _SC_SUBCORES


def _topk_body(n_real, qb, q_ref, m_ref, vals_ref, idx_ref, runv_ref, runi_ref):
    ni = pl.program_id(1)
    nblocks = pl.num_programs(1)

    @pl.when(ni == 0)
    def _init():
        runv_ref[...] = jnp.full((qb, 128), _NEG_INF, jnp.float32)
        runi_ref[...] = jnp.zeros((qb, 128), jnp.float32)

    q = q_ref[...]
    qn = q / jnp.maximum(
        jnp.sqrt(jnp.sum(q * q, axis=1, keepdims=True)), 1e-12)
    m = m_ref[...]
    mn = m / jnp.maximum(
        jnp.sqrt(jnp.sum(m * m, axis=1, keepdims=True)), 1e-12)
    s = lax.dot_general(qn, mn, (((1,), (1,)), ((), ())))

    base = ni * NB
    colf = (lax.broadcasted_iota(jnp.int32, s.shape, 1).astype(jnp.float32)
            + base.astype(jnp.float32))
    s = jnp.where(colf < float(n_real), s, _NEG_INF)

    ss = jnp.concatenate([runv_ref[...], s], axis=1)
    ii = jnp.concatenate([runi_ref[...], colf], axis=1)
    vals, gidx = [], []
    for _ in range(K):
        mx = jnp.max(ss, axis=1, keepdims=True)
        t = jnp.where(ss == mx, ii, _BIGF)
        gi = jnp.min(t, axis=1, keepdims=True)
        ss = jnp.where(t == gi, _NEG_INF, ss)
        vals.append(mx)
        gidx.append(gi)
    nv = jnp.concatenate(vals, axis=1)
    ngi = jnp.concatenate(gidx, axis=1)
    runv_ref[...] = jnp.concatenate(
        [nv, jnp.full((qb, 128 - K), _NEG_INF, jnp.float32)], axis=1)
    runi_ref[...] = jnp.concatenate(
        [ngi, jnp.zeros((qb, 128 - K), jnp.float32)], axis=1)

    @pl.when(ni == nblocks - 1)
    def _emit():
        vals_ref[...] = nv
        idx_ref[...] = ngi.astype(jnp.int32)


def _topk_scores(query, memory_states):
    q, d = query.shape
    n = memory_states.shape[0]
    qb = QB if q % QB == 0 else (8 if q % 8 == 0 else 1)
    n_pad = ((n + NB - 1) // NB) * NB
    if n_pad != n:
        memory_states = jnp.pad(memory_states, ((0, n_pad - n), (0, 0)))
    grid = (q // qb, n_pad // NB)
    vals, idx = pl.pallas_call(
        functools.partial(_topk_body, n, qb),
        grid=grid,
        in_specs=[
            pl.BlockSpec((qb, d), lambda qi, ni: (qi, 0)),
            pl.BlockSpec((NB, d), lambda qi, ni: (ni, 0)),
        ],
        out_specs=[
            pl.BlockSpec((qb, K), lambda qi, ni: (qi, 0)),
            pl.BlockSpec((qb, K), lambda qi, ni: (qi, 0)),
        ],
        out_shape=[
            jax.ShapeDtypeStruct((q, K), jnp.float32),
            jax.ShapeDtypeStruct((q, K), jnp.int32),
        ],
        scratch_shapes=[
            pltpu.VMEM((qb, 128), jnp.float32),
            pltpu.VMEM((qb, 128), jnp.float32),
        ],
    )(query, memory_states)
    return vals, idx


def _sc_gather(table, flat_idx):
    b, d = flat_idx.shape[0], table.shape[1]
    b_per_w = b // _SC_WORKERS
    mesh = plsc.VectorSubcoreMesh(core_axis_name="c", subcore_axis_name="s")

    @functools.partial(
        pl.kernel,
        mesh=mesh,
        out_type=jax.ShapeDtypeStruct((b, d), jnp.float32),
        scratch_types=[
            pltpu.VMEM((b_per_w,), jnp.int32),
            pltpu.VMEM((b_per_w, d), jnp.float32),
            pltpu.SemaphoreType.DMA,
        ],
    )
    def gather(table_hbm, idx_hbm, out_hbm, idx_v, rows_v, sem):
        wid = lax.axis_index("s") * _SC_CORES + lax.axis_index("c")
        base = wid * b_per_w
        pltpu.sync_copy(idx_hbm.at[pl.ds(base, b_per_w)], idx_v)
        pltpu.async_copy(table_hbm.at[idx_v], rows_v, sem).wait()
        pltpu.sync_copy(rows_v, out_hbm.at[pl.ds(base, b_per_w)])

    return gather(table, flat_idx)


def kernel(query, memory_states, k):
    if query.ndim == 1:
        query = query[None, :]
    q, d = query.shape
    vals, idx = _topk_scores(query, memory_states)
    flat_idx = idx.reshape(q * K)
    algn = 8 * _SC_WORKERS
    b_pad = ((q * K + algn - 1) // algn) * algn
    if b_pad != q * K:
        flat_idx = jnp.pad(flat_idx, (0, b_pad - q * K))
    rows = _sc_gather(memory_states, flat_idx)
    retrieved = rows[: q * K].reshape(q, K, d)
    return (retrieved, vals)


# speedup vs baseline: 2.7152x; 1.1217x over previous
import functools

import jax
import jax.numpy as jnp
from jax import lax
from jax.experimental import pallas as pl
from jax.experimental.pallas import tpu as pltpu
from jax.experimental.pallas import tpu_sc as plsc

K = 8
QB = 512
NB = 4096
_NEG_INF = float("-inf")
_BIGF = 1e9

_SC_CORES = 2
_SC_SUBCORES = 16
_SC_WORKERS = _SC_CORES * _SC_SUBCORES


def _topk_body(n_real, qb, q_ref, m_ref, vals_ref, idx_ref, runv_ref, runi_ref):
    ni = pl.program_id(1)
    nblocks = pl.num_programs(1)

    @pl.when(ni == 0)
    def _init():
        runv_ref[...] = jnp.full((qb, 128), _NEG_INF, jnp.float32)
        runi_ref[...] = jnp.zeros((qb, 128), jnp.float32)

    q = q_ref[...]
    qn = q / jnp.maximum(
        jnp.sqrt(jnp.sum(q * q, axis=1, keepdims=True)), 1e-12)
    m = m_ref[...]
    mn = m / jnp.maximum(
        jnp.sqrt(jnp.sum(m * m, axis=1, keepdims=True)), 1e-12)
    s = lax.dot_general(qn, mn, (((1,), (1,)), ((), ())))

    base = ni * NB
    colf = (lax.broadcasted_iota(jnp.int32, s.shape, 1).astype(jnp.float32)
            + base.astype(jnp.float32))
    s = jnp.where(colf < float(n_real), s, _NEG_INF)

    ss = jnp.concatenate([runv_ref[...], s], axis=1)
    ii = jnp.concatenate([runi_ref[...], colf], axis=1)
    vals, gidx = [], []
    for _ in range(K):
        mx = jnp.max(ss, axis=1, keepdims=True)
        t = jnp.where(ss == mx, ii, _BIGF)
        gi = jnp.min(t, axis=1, keepdims=True)
        ss = jnp.where(t == gi, _NEG_INF, ss)
        vals.append(mx)
        gidx.append(gi)
    nv = jnp.concatenate(vals, axis=1)
    ngi = jnp.concatenate(gidx, axis=1)
    runv_ref[...] = jnp.concatenate(
        [nv, jnp.full((qb, 128 - K), _NEG_INF, jnp.float32)], axis=1)
    runi_ref[...] = jnp.concatenate(
        [ngi, jnp.zeros((qb, 128 - K), jnp.float32)], axis=1)

    @pl.when(ni == nblocks - 1)
    def _emit():
        vals_ref[...] = nv
        idx_ref[...] = ngi.astype(jnp.int32)


def _topk_scores(query, memory_states):
    q, d = query.shape
    n = memory_states.shape[0]
    qb = QB if q % QB == 0 else (8 if q % 8 == 0 else 1)
    grid = (q // qb, (n + NB - 1) // NB)
    vals, idx = pl.pallas_call(
        functools.partial(_topk_body, n, qb),
        grid=grid,
        in_specs=[
            pl.BlockSpec((qb, d), lambda qi, ni: (qi, 0)),
            pl.BlockSpec((NB, d), lambda qi, ni: (ni, 0)),
        ],
        out_specs=[
            pl.BlockSpec((qb, K), lambda qi, ni: (qi, 0)),
            pl.BlockSpec((qb, K), lambda qi, ni: (qi, 0)),
        ],
        out_shape=[
            jax.ShapeDtypeStruct((q, K), jnp.float32),
            jax.ShapeDtypeStruct((q, K), jnp.int32),
        ],
        scratch_shapes=[
            pltpu.VMEM((qb, 128), jnp.float32),
            pltpu.VMEM((qb, 128), jnp.float32),
        ],
    )(query, memory_states)
    return vals, idx


def _sc_gather(table, flat_idx):
    b, d = flat_idx.shape[0], table.shape[1]
    b_per_w = b // _SC_WORKERS
    mesh = plsc.VectorSubcoreMesh(core_axis_name="c", subcore_axis_name="s")

    @functools.partial(
        pl.kernel,
        mesh=mesh,
        out_type=jax.ShapeDtypeStruct((b, d), jnp.float32),
        scratch_types=[
            pltpu.VMEM((b_per_w,), jnp.int32),
            pltpu.VMEM((b_per_w, d), jnp.float32),
            pltpu.SemaphoreType.DMA,
        ],
    )
    def gather(table_hbm, idx_hbm, out_hbm, idx_v, rows_v, sem):
        wid = lax.axis_index("s") * _SC_CORES + lax.axis_index("c")
        base = wid * b_per_w
        pltpu.sync_copy(idx_hbm.at[pl.ds(base, b_per_w)], idx_v)
        pltpu.async_copy(table_hbm.at[idx_v], rows_v, sem).wait()
        pltpu.sync_copy(rows_v, out_hbm.at[pl.ds(base, b_per_w)])

    return gather(table, flat_idx)


def kernel(query, memory_states, k):
    if query.ndim == 1:
        query = query[None, :]
    q, d = query.shape
    vals, idx = _topk_scores(query, memory_states)
    flat_idx = idx.reshape(q * K)
    algn = 8 * _SC_WORKERS
    b_pad = ((q * K + algn - 1) // algn) * algn
    if b_pad != q * K:
        flat_idx = jnp.pad(flat_idx, (0, b_pad - q * K))
    rows = _sc_gather(memory_states, flat_idx)
    retrieved = rows[: q * K].reshape(q, K, d)
    return (retrieved, vals)
